# Initial kernel scaffold; baseline (speedup 1.0000x reference)
import jax
import jax.numpy as jnp
from jax.experimental import pallas as pl


def kernel(token_type_ids, W1, W2):
    raise NotImplementedError("write your pallas kernel here")


# speedup vs baseline: 1.4086x; 1.4086x over previous
import functools

import jax
import jax.numpy as jnp
from jax import lax
from jax.experimental import pallas as pl
from jax.experimental.pallas import tpu as pltpu
from jax.experimental.pallas import tpu_sc as plsc

HIDDEN = 1024
B, S = 4, 8192
N = B * S
NC, NS = 2, 16
NW = NC * NS
TPW = N // NW
C = 16
NCH = TPW // C
LANES = 16


def _emb_body(idx1_hbm, idx2_hbm, w1_hbm, w2_hbm, out_hbm,
              idx1_v, idx2_v, buf_a, buf_b, sem_a, sem_b):
    wid = lax.axis_index("s") * NC + lax.axis_index("c")
    base = wid * TPW
    pltpu.sync_copy(idx1_hbm.at[wid], idx1_v)
    pltpu.sync_copy(idx2_hbm.at[wid], idx2_v)

    def chunk(j, carry):
        cp_a = pltpu.async_copy(w1_hbm.at[idx1_v.at[j]], buf_a, sem_a)
        cp_b = pltpu.async_copy(w2_hbm.at[idx2_v.at[j]], buf_b, sem_b)
        cp_a.wait()
        cp_b.wait()

        def row(r, rc):
            for cc in range(HIDDEN // LANES):
                sl = pl.ds(cc * LANES, LANES)
                buf_a[r, sl] = buf_a[r, sl] + buf_b[r, sl]
            return rc

        lax.fori_loop(0, C, row, 0)
        pltpu.sync_copy(buf_a, out_hbm.at[pl.ds(base + j * C, C)])
        return carry

    lax.fori_loop(0, NCH, chunk, 0)


_emb = functools.partial(
    pl.kernel,
    mesh=plsc.VectorSubcoreMesh(core_axis_name="c", subcore_axis_name="s"),
    out_type=jax.ShapeDtypeStruct((N, HIDDEN), jnp.float32),
    scratch_types=[
        pltpu.VMEM((NCH, C), jnp.int32),
        pltpu.VMEM((NCH, C), jnp.int32),
        pltpu.VMEM((C, HIDDEN), jnp.float32),
        pltpu.VMEM((C, HIDDEN), jnp.float32),
        pltpu.SemaphoreType.DMA,
        pltpu.SemaphoreType.DMA,
    ],
)(_emb_body)


def kernel(token_type_ids, W1, W2):
    ids = token_type_ids.astype(jnp.int32)
    idx1 = ids[:, :, 1].reshape(NW, NCH, C)
    idx2 = ids[:, :, 2].reshape(NW, NCH, C)
    out = _emb(idx1, idx2, W1, W2)
    return out.reshape(B, S, HIDDEN)


# speedup vs baseline: 1.7187x; 1.2201x over previous
import functools

import jax
import jax.numpy as jnp
from jax import lax
from jax.experimental import pallas as pl
from jax.experimental.pallas import tpu as pltpu
from jax.experimental.pallas import tpu_sc as plsc

HIDDEN = 1024
B, S = 4, 8192
N = B * S
NC, NS = 2, 16
NW = NC * NS
TPW = N // NW
C = 16
NCH = TPW // C
LANES = 16
NBUF = 2


def _emb_body(idx1_hbm, idx2_hbm, w1_hbm, w2_hbm, out_hbm,
              idx1_v, idx2_v, bufs_a, bufs_b, sem_g0, sem_g1, sem_s0, sem_s1):
    wid = lax.axis_index("s") * NC + lax.axis_index("c")
    base = wid * TPW
    pltpu.sync_copy(idx1_hbm.at[wid], idx1_v)
    pltpu.sync_copy(idx2_hbm.at[wid], idx2_v)
    sem_g = (sem_g0, sem_g1)
    sem_s = (sem_s0, sem_s1)

    def start_gathers(j, slot):
        pltpu.async_copy(w1_hbm.at[idx1_v.at[j]], bufs_a.at[slot], sem_g[slot])
        pltpu.async_copy(w2_hbm.at[idx2_v.at[j]], bufs_b.at[slot], sem_g[slot])

    def wait_gathers(slot):
        pltpu.make_async_copy(w1_hbm.at[idx1_v.at[0]], bufs_a.at[slot],
                              sem_g[slot]).wait()
        pltpu.make_async_copy(w2_hbm.at[idx2_v.at[0]], bufs_b.at[slot],
                              sem_g[slot]).wait()

    def wait_store(slot):
        pltpu.make_async_copy(bufs_a.at[slot],
                              out_hbm.at[pl.ds(base, C)], sem_s[slot]).wait()

    start_gathers(0, 0)

    def outer(jj, carry):
        for b in range(NBUF):
            j = jj * NBUF + b
            nb = 1 - b

            @pl.when(j + 1 < NCH)
            def _():
                start_gathers(j + 1, nb)

            wait_gathers(b)

            def row(r, rc):
                for cc in range(HIDDEN // LANES):
                    sl = pl.ds(cc * LANES, LANES)
                    plsc.addupdate(bufs_a.at[b, r, sl], bufs_b[b, r, sl])
                return rc

            lax.fori_loop(0, C, row, 0)

            @pl.when(j >= NBUF)
            def _():
                wait_store(b)

            pltpu.async_copy(bufs_a.at[b],
                             out_hbm.at[pl.ds(base + j * C, C)], sem_s[b])
        return carry

    lax.fori_loop(0, NCH // NBUF, outer, 0)
    wait_store(0)
    wait_store(1)


_emb = functools.partial(
    pl.kernel,
    mesh=plsc.VectorSubcoreMesh(core_axis_name="c", subcore_axis_name="s"),
    out_type=jax.ShapeDtypeStruct((N, HIDDEN), jnp.float32),
    scratch_types=[
        pltpu.VMEM((NCH, C), jnp.int32),
        pltpu.VMEM((NCH, C), jnp.int32),
        pltpu.VMEM((NBUF, C, HIDDEN), jnp.float32),
        pltpu.VMEM((NBUF, C, HIDDEN), jnp.float32),
        pltpu.SemaphoreType.DMA,
        pltpu.SemaphoreType.DMA,
        pltpu.SemaphoreType.DMA,
        pltpu.SemaphoreType.DMA,
    ],
)(_emb_body)


def kernel(token_type_ids, W1, W2):
    ids = token_type_ids.astype(jnp.int32)
    idx1 = ids[:, :, 1].reshape(NW, NCH, C)
    idx2 = ids[:, :, 2].reshape(NW, NCH, C)
    out = _emb(idx1, idx2, W1, W2)
    return out.reshape(B, S, HIDDEN)


# speedup vs baseline: 2.0097x; 1.1693x over previous
import functools

import jax
import jax.numpy as jnp
from jax import lax
from jax.experimental import pallas as pl
from jax.experimental.pallas import tpu as pltpu
from jax.experimental.pallas import tpu_sc as plsc

HIDDEN = 1024
B, S = 4, 8192
N = B * S
NC, NS = 2, 16
NW = NC * NS
TPW = N // NW
C = 16
NCH = TPW // C
LANES = 16
NBUF = 2


def _emb_body(idx1_hbm, idx2_hbm, w1_hbm, w2_hbm, out_hbm,
              idx1_v, idx2_v, bufs_a, bufs_b, sem_g0, sem_g1, sem_s0, sem_s1):
    wid = lax.axis_index("s") * NC + lax.axis_index("c")
    base = wid * TPW
    pltpu.sync_copy(idx1_hbm.at[wid], idx1_v)
    pltpu.sync_copy(idx2_hbm.at[wid], idx2_v)
    sem_g = (sem_g0, sem_g1)
    sem_s = (sem_s0, sem_s1)

    def start_gathers(j, slot):
        pltpu.async_copy(w1_hbm.at[idx1_v.at[j]], bufs_a.at[slot], sem_g[slot])
        pltpu.async_copy(w2_hbm.at[idx2_v.at[j]], bufs_b.at[slot], sem_g[slot])

    def wait_gathers(slot):
        pltpu.make_async_copy(w1_hbm.at[idx1_v.at[0]], bufs_a.at[slot],
                              sem_g[slot]).wait()
        pltpu.make_async_copy(w2_hbm.at[idx2_v.at[0]], bufs_b.at[slot],
                              sem_g[slot]).wait()

    def wait_store(slot):
        pltpu.make_async_copy(bufs_a.at[slot],
                              out_hbm.at[pl.ds(base, C)], sem_s[slot]).wait()

    start_gathers(0, 0)

    def outer(jj, carry):
        for b in range(NBUF):
            j = jj * NBUF + b
            nb = 1 - b

            @pl.when(j + 1 < NCH)
            def _():
                start_gathers(j + 1, nb)

            wait_gathers(b)

            def row(r, rc):
                for cc in range(HIDDEN // LANES):
                    sl = pl.ds(cc * LANES, LANES)
                    bufs_a[b, r, sl] = bufs_a[b, r, sl] + bufs_b[b, r, sl]
                return rc

            lax.fori_loop(0, C, row, 0)

            @pl.when(j >= NBUF)
            def _():
                wait_store(b)

            pltpu.async_copy(bufs_a.at[b],
                             out_hbm.at[pl.ds(base + j * C, C)], sem_s[b])
        return carry

    lax.fori_loop(0, NCH // NBUF, outer, 0)
    wait_store(0)
    wait_store(1)


_emb = functools.partial(
    pl.kernel,
    mesh=plsc.VectorSubcoreMesh(core_axis_name="c", subcore_axis_name="s"),
    out_type=jax.ShapeDtypeStruct((N, HIDDEN), jnp.float32),
    scratch_types=[
        pltpu.VMEM((NCH, C), jnp.int32),
        pltpu.VMEM((NCH, C), jnp.int32),
        pltpu.VMEM((NBUF, C, HIDDEN), jnp.float32),
        pltpu.VMEM((NBUF, C, HIDDEN), jnp.float32),
        pltpu.SemaphoreType.DMA,
        pltpu.SemaphoreType.DMA,
        pltpu.SemaphoreType.DMA,
        pltpu.SemaphoreType.DMA,
    ],
)(_emb_body)


def kernel(token_type_ids, W1, W2):
    ids = token_type_ids.astype(jnp.int32)
    idx1 = ids[:, :, 1].reshape(NW, NCH, C)
    idx2 = ids[:, :, 2].reshape(NW, NCH, C)
    out = _emb(idx1, idx2, W1, W2)
    return out.reshape(B, S, HIDDEN)
